# baseline (speedup 1.0000x reference)
import functools

import jax
import jax.numpy as jnp
from jax import lax
from jax.experimental import pallas as pl
from jax.experimental.pallas import tpu as pltpu
from jax.experimental.pallas import tpu_sc as plsc

_EMB = 102
_VOCABS = (6, 6, 3, 2, 4)
_DIVS = (144, 24, 8, 4, 1)
_NUM_COMB = 864
_NUM_PAD = 896
_OUT = 512
_BATCH = 16384

_NC = 2
_NS = 16
_NW = _NC * _NS
_BPW = _BATCH // _NW
_CHUNK = 32
_NBUF = 6
_PRIME = 3
_NCH = _BPW // _CHUNK
_LANES = 16


def _table_body(tg, tm, tt, tk, ts, w0, w1, w2, w3, w4, b2d, out_ref):
    tabs = (tg, tm, tt, tk, ts)
    wparts = (w0, w1, w2, w3, w4)
    cid = lax.broadcasted_iota(jnp.int32, (_NUM_PAD, 1), 0)
    acc = jnp.zeros((_NUM_PAD, _OUT), jnp.float32)
    for a in range(5):
        vocab, div = _VOCABS[a], _DIVS[a]
        sel = (cid // div) % vocab
        oh = (sel == lax.broadcasted_iota(jnp.int32, (_NUM_PAD, vocab), 1))
        p = jnp.dot(tabs[a][...], wparts[a][...],
                    preferred_element_type=jnp.float32)
        acc = acc + jnp.dot(oh.astype(jnp.float32), p,
                            preferred_element_type=jnp.float32)
    out_ref[...] = acc + b2d[...]


_build_table = pl.pallas_call(
    _table_body,
    out_shape=jax.ShapeDtypeStruct((_NUM_PAD, _OUT), jnp.float32),
)


def _gather_body(table_hbm, g_hbm, m_hbm, t_hbm, k_hbm, s_hbm, out_hbm,
                 gv, mv, tv, kv, sv, cidx, bufs_ref, gsems, wsems):
    wid = lax.axis_index("s") * _NC + lax.axis_index("c")
    base = wid * _BPW
    pltpu.sync_copy(g_hbm.at[pl.ds(base, _BPW)], gv)
    pltpu.sync_copy(m_hbm.at[pl.ds(base, _BPW)], mv)
    pltpu.sync_copy(t_hbm.at[pl.ds(base, _BPW)], tv)
    pltpu.sync_copy(k_hbm.at[pl.ds(base, _BPW)], kv)
    pltpu.sync_copy(s_hbm.at[pl.ds(base, _BPW)], sv)

    def cbody(j, carry):
        off = j * _LANES
        c = (gv[pl.ds(off, _LANES)] * _DIVS[0]
             + mv[pl.ds(off, _LANES)] * _DIVS[1]
             + tv[pl.ds(off, _LANES)] * _DIVS[2]
             + kv[pl.ds(off, _LANES)] * _DIVS[3]
             + sv[pl.ds(off, _LANES)])
        cidx[pl.ds(off, _LANES)] = c
        return carry

    lax.fori_loop(0, _BPW // _LANES, cbody, 0)

    def issue_gather(c):
        b = c % _NBUF
        return pltpu.async_copy(
            table_hbm.at[cidx.at[pl.ds(c * _CHUNK, _CHUNK)]],
            bufs_ref.at[b], gsems.at[b])

    def issue_write(c):
        b = c % _NBUF
        return pltpu.async_copy(
            bufs_ref.at[b], out_hbm.at[pl.ds(base + c * _CHUNK, _CHUNK)],
            wsems.at[b])

    gh = {c: issue_gather(c) for c in range(_PRIME)}
    wh = {}
    waited = set()
    for c in range(_NCH):
        gh[c].wait()
        wh[c] = issue_write(c)
        nxt = c + _PRIME
        if nxt < _NCH:
            prev = nxt - _NBUF
            if prev >= 0:
                wh[prev].wait()
                waited.add(prev)
            gh[nxt] = issue_gather(nxt)
    for c in range(_NCH):
        if c not in waited:
            wh[c].wait()


@functools.lru_cache(maxsize=None)
def _make_gather():
    return pl.kernel(
        _gather_body,
        out_type=jax.ShapeDtypeStruct((_BATCH, _OUT), jnp.float32),
        mesh=plsc.VectorSubcoreMesh(core_axis_name="c", subcore_axis_name="s"),
        scratch_types=[
            pltpu.VMEM((_BPW,), jnp.int32),
            pltpu.VMEM((_BPW,), jnp.int32),
            pltpu.VMEM((_BPW,), jnp.int32),
            pltpu.VMEM((_BPW,), jnp.int32),
            pltpu.VMEM((_BPW,), jnp.int32),
            pltpu.VMEM((_BPW,), jnp.int32),
            pltpu.VMEM((_NBUF, _CHUNK, _OUT), jnp.float32),
            pltpu.SemaphoreType.DMA((_NBUF,)),
            pltpu.SemaphoreType.DMA((_NBUF,)),
        ],
    )


def kernel(genre, mood, tempo, key_mode, time_signature,
           emb_genre, emb_mood, emb_tempo, emb_key_mode, emb_time_signature,
           W, b):
    wparts = [W[a * _EMB:(a + 1) * _EMB, :] for a in range(5)]
    table = _build_table(emb_genre, emb_mood, emb_tempo, emb_key_mode,
                         emb_time_signature, *wparts, b.reshape(1, _OUT))
    idxs = [jnp.asarray(x, jnp.int32)
            for x in (genre, mood, tempo, key_mode, time_signature)]
    return _make_gather()(table, *idxs)


# speedup vs baseline: 1.0261x; 1.0261x over previous
import functools

import jax
import jax.numpy as jnp
from jax import lax
from jax.experimental import pallas as pl
from jax.experimental.pallas import tpu as pltpu
from jax.experimental.pallas import tpu_sc as plsc

_EMB = 102
_VOCABS = (6, 6, 3, 2, 4)
_DIVS = (144, 24, 8, 4, 1)
_NUM_COMB = 864
_NUM_PAD = 896
_OUT = 512
_BATCH = 16384

_NC = 2
_NS = 16
_NW = _NC * _NS
_BPW = _BATCH // _NW
_CHUNK = 32
_NBUF = 6
_PRIME = 3
_NCH = _BPW // _CHUNK
_LANES = 16


def _table_body(tg, tm, tt, tk, ts, w_ref, b2d, out_ref):
    tabs = (tg, tm, tt, tk, ts)
    w = w_ref[...]
    cid = lax.broadcasted_iota(jnp.int32, (_NUM_PAD, 1), 0)
    acc = jnp.zeros((_NUM_PAD, _OUT), jnp.float32)
    for a in range(5):
        vocab, div = _VOCABS[a], _DIVS[a]
        sel = (cid // div) % vocab
        oh = (sel == lax.broadcasted_iota(jnp.int32, (_NUM_PAD, vocab), 1))
        p = jnp.dot(tabs[a][...], lax.slice(w, (a * _EMB, 0), ((a + 1) * _EMB, _OUT)),
                    preferred_element_type=jnp.float32)
        acc = acc + jnp.dot(oh.astype(jnp.float32), p,
                            preferred_element_type=jnp.float32)
    out_ref[...] = acc + b2d[...]


_build_table = pl.pallas_call(
    _table_body,
    out_shape=jax.ShapeDtypeStruct((_NUM_PAD, _OUT), jnp.float32),
)


def _gather_body(table_hbm, g_hbm, m_hbm, t_hbm, k_hbm, s_hbm, out_hbm,
                 gv, mv, tv, kv, sv, cidx, bufs_ref, gsems, wsems):
    wid = lax.axis_index("s") * _NC + lax.axis_index("c")
    base = wid * _BPW
    pltpu.sync_copy(g_hbm.at[pl.ds(base, _BPW)], gv)
    pltpu.sync_copy(m_hbm.at[pl.ds(base, _BPW)], mv)
    pltpu.sync_copy(t_hbm.at[pl.ds(base, _BPW)], tv)
    pltpu.sync_copy(k_hbm.at[pl.ds(base, _BPW)], kv)
    pltpu.sync_copy(s_hbm.at[pl.ds(base, _BPW)], sv)

    def cbody(j, carry):
        off = j * _LANES
        c = (gv[pl.ds(off, _LANES)] * _DIVS[0]
             + mv[pl.ds(off, _LANES)] * _DIVS[1]
             + tv[pl.ds(off, _LANES)] * _DIVS[2]
             + kv[pl.ds(off, _LANES)] * _DIVS[3]
             + sv[pl.ds(off, _LANES)])
        cidx[pl.ds(off, _LANES)] = c
        return carry

    lax.fori_loop(0, _BPW // _LANES, cbody, 0)

    def issue_gather(c):
        b = c % _NBUF
        return pltpu.async_copy(
            table_hbm.at[cidx.at[pl.ds(c * _CHUNK, _CHUNK)]],
            bufs_ref.at[b], gsems.at[b])

    def issue_write(c):
        b = c % _NBUF
        return pltpu.async_copy(
            bufs_ref.at[b], out_hbm.at[pl.ds(base + c * _CHUNK, _CHUNK)],
            wsems.at[b])

    gh = {c: issue_gather(c) for c in range(_PRIME)}
    wh = {}
    waited = set()
    for c in range(_NCH):
        gh[c].wait()
        wh[c] = issue_write(c)
        nxt = c + _PRIME
        if nxt < _NCH:
            prev = nxt - _NBUF
            if prev >= 0:
                wh[prev].wait()
                waited.add(prev)
            gh[nxt] = issue_gather(nxt)
    for c in range(_NCH):
        if c not in waited:
            wh[c].wait()


@functools.lru_cache(maxsize=None)
def _make_gather():
    return pl.kernel(
        _gather_body,
        out_type=jax.ShapeDtypeStruct((_BATCH, _OUT), jnp.float32),
        mesh=plsc.VectorSubcoreMesh(core_axis_name="c", subcore_axis_name="s"),
        scratch_types=[
            pltpu.VMEM((_BPW,), jnp.int32),
            pltpu.VMEM((_BPW,), jnp.int32),
            pltpu.VMEM((_BPW,), jnp.int32),
            pltpu.VMEM((_BPW,), jnp.int32),
            pltpu.VMEM((_BPW,), jnp.int32),
            pltpu.VMEM((_BPW,), jnp.int32),
            pltpu.VMEM((_NBUF, _CHUNK, _OUT), jnp.float32),
            pltpu.SemaphoreType.DMA((_NBUF,)),
            pltpu.SemaphoreType.DMA((_NBUF,)),
        ],
    )


def kernel(genre, mood, tempo, key_mode, time_signature,
           emb_genre, emb_mood, emb_tempo, emb_key_mode, emb_time_signature,
           W, b):
    table = _build_table(emb_genre, emb_mood, emb_tempo, emb_key_mode,
                         emb_time_signature, W, b.reshape(1, _OUT))
    idxs = [jnp.asarray(x, jnp.int32)
            for x in (genre, mood, tempo, key_mode, time_signature)]
    return _make_gather()(table, *idxs)
